# baseline (speedup 1.0000x reference)
import functools

import jax
import jax.numpy as jnp
from jax import lax
from jax.experimental import pallas as pl
from jax.experimental.pallas import tpu as pltpu
from jax.experimental.pallas import tpu_sc as plsc

N = 10000
E = 320000
D = 128
S = 4
T = 8
K = 16

NPAD = 10240
NCORES = 2
NSUB = 16
NTILES = NCORES * NSUB
CHUNK = 128
CHUNKS = 80
HCHUNKS = CHUNKS // 2
EPT = CHUNKS * CHUNK
RPT = NPAD // NSUB
DEGC = 16
RB = 512
NB = NPAD // RB


@functools.cache
def _mesh():
    return plsc.VectorSubcoreMesh(core_axis_name="c", subcore_axis_name="s",
                                  num_cores=NCORES, num_subcores=NSUB)


def _spmm_sc(y, gat, sca):
    @functools.partial(
        pl.kernel,
        out_type=jax.ShapeDtypeStruct((NCORES, NPAD, D), jnp.float32),
        mesh=_mesh(),
        scratch_types=[
            pltpu.VMEM((HCHUNKS, CHUNK), jnp.int32),
            pltpu.VMEM((HCHUNKS, CHUNK), jnp.int32),
            pltpu.VMEM((CHUNK, D), jnp.float32),
            pltpu.VMEM((CHUNK, D), jnp.float32),
            pltpu.VMEM_SHARED((NPAD, D), jnp.float32),
            pltpu.SemaphoreType.DMA,
            pltpu.SemaphoreType.DMA,
        ],
    )
    def k(y_hbm, gat_hbm, sca_hbm, out_hbm,
          gat_v, sca_v, buf0, buf1, agg, sem0, sem1):
        cid = lax.axis_index("c")
        sid = lax.axis_index("s")

        zero16 = jnp.zeros((16,), jnp.float32)

        def zbody(r, carry):
            for c in range(D // 16):
                buf0[r, pl.ds(c * 16, 16)] = zero16
            return carry

        lax.fori_loop(0, CHUNK, zbody, 0)
        row0 = sid * RPT
        for z in range(RPT // CHUNK):
            pltpu.sync_copy(buf0, agg.at[pl.ds(row0 + z * CHUNK, CHUNK)])
        plsc.subcore_barrier()

        for h in range(2):
            pltpu.sync_copy(gat_hbm.at[cid, sid, pl.ds(h * HCHUNKS, HCHUNKS)],
                            gat_v)
            pltpu.sync_copy(sca_hbm.at[cid, sid, pl.ds(h * HCHUNKS, HCHUNKS)],
                            sca_v)
            pltpu.async_copy(y_hbm.at[gat_v.at[0]], buf0, sem0)

            def step(i, carry):
                g = i * 2
                pltpu.async_copy(y_hbm.at[gat_v.at[g + 1]], buf1, sem1)
                pltpu.make_async_copy(y_hbm.at[gat_v.at[g]], buf0, sem0).wait()
                pltpu.sync_copy(buf0, agg.at[sca_v.at[g]], add=True)

                @pl.when(g + 2 < HCHUNKS)
                def _():
                    pltpu.async_copy(y_hbm.at[gat_v.at[g + 2]], buf0, sem0)

                pltpu.make_async_copy(y_hbm.at[gat_v.at[g + 1]], buf1,
                                      sem1).wait()
                pltpu.sync_copy(buf1, agg.at[sca_v.at[g + 1]], add=True)
                return carry

            lax.fori_loop(0, HCHUNKS // 2, step, 0)
        plsc.subcore_barrier()
        pltpu.sync_copy(agg.at[pl.ds(row0, RPT)],
                        out_hbm.at[cid, pl.ds(row0, RPT)])

    return k(y, gat, sca)


def _invdeg_tc(dp):
    def body(p_ref, o_ref):
        d = p_ref[0] + p_ref[1]
        inv = 1.0 / jnp.maximum(d[:, :1], 1.0)
        o_ref[...] = jnp.broadcast_to(inv, (RB, D))

    return pl.pallas_call(
        body,
        grid=(NB,),
        in_specs=[pl.BlockSpec((NCORES, RB, D), lambda j: (0, j, 0))],
        out_specs=pl.BlockSpec((RB, D), lambda j: (j, 0)),
        out_shape=jax.ShapeDtypeStruct((NPAD, D), jnp.float32),
    )(dp)


def _combine_tc(p, invb):
    def body(p_ref, i_ref, o_ref):
        o_ref[...] = (p_ref[0] + p_ref[1]) * i_ref[...]

    return pl.pallas_call(
        body,
        grid=(NB,),
        in_specs=[pl.BlockSpec((NCORES, RB, D), lambda j: (0, j, 0)),
                  pl.BlockSpec((RB, D), lambda j: (j, 0))],
        out_specs=pl.BlockSpec((RB, D), lambda j: (j, 0)),
        out_shape=jax.ShapeDtypeStruct((NPAD, D), jnp.float32),
    )(p, invb)


def _acc_tc(ys, coeffs):
    def body(c_ref, *refs):
        y_refs = refs[:T + 1]
        o_ref = refs[T + 1]
        s = pl.program_id(0)
        logits = [c_ref[s, t] for t in range(T + 1)]
        m = logits[0]
        for t in range(1, T + 1):
            m = jnp.maximum(m, logits[t])
        es = [jnp.exp(l - m) for l in logits]
        tot = es[0]
        for t in range(1, T + 1):
            tot = tot + es[t]
        acc = (es[0] / tot) * y_refs[0][...]
        for t in range(1, T + 1):
            acc = acc + (es[t] / tot) * y_refs[t][...]
        o_ref[0] = acc

    in_specs = [pl.BlockSpec(memory_space=pltpu.SMEM)] + \
               [pl.BlockSpec((RB, D), lambda s, j: (j, 0))] * (T + 1)
    return pl.pallas_call(
        body,
        grid=(S, NB),
        in_specs=in_specs,
        out_specs=pl.BlockSpec((1, RB, D), lambda s, j: (s, j, 0)),
        out_shape=jax.ShapeDtypeStruct((S, NPAD, D), jnp.float32),
    )(coeffs, *ys)


def _mean_tc(acc):
    def body(a_ref, o_ref):
        j = pl.program_id(1)
        rows = j * RB + lax.broadcasted_iota(jnp.int32, (RB, D), 0)
        blk = jnp.where(rows < N, a_ref[0], 0.0)
        part = jnp.sum(blk, axis=0, keepdims=True)[None]

        @pl.when(j == 0)
        def _():
            o_ref[...] = jnp.zeros_like(o_ref)

        o_ref[...] += part

        @pl.when(j == NB - 1)
        def _():
            o_ref[...] = o_ref[...] / float(N)

    return pl.pallas_call(
        body,
        grid=(S, NB),
        in_specs=[pl.BlockSpec((1, RB, D), lambda s, j: (s, j, 0))],
        out_specs=pl.BlockSpec((1, 1, D), lambda s, j: (s, 0, 0)),
        out_shape=jax.ShapeDtypeStruct((S, 1, D), jnp.float32),
    )(acc)


def _cov_tc(acc, mean):
    def body(a_ref, m_ref, o_ref):
        j = pl.program_id(1)
        yc = a_ref[0] - m_ref[0]
        rows = j * RB + lax.broadcasted_iota(jnp.int32, (RB, D), 0)
        yc = jnp.where(rows < N, yc, 0.0)
        c = lax.dot_general(yc, yc, (((0,), (0,)), ((), ())),
                            preferred_element_type=jnp.float32)

        @pl.when(j == 0)
        def _():
            o_ref[...] = jnp.zeros_like(o_ref)

        o_ref[...] += c[None]

        @pl.when(j == NB - 1)
        def _():
            o_ref[...] = o_ref[...] / float(N - 1)

    return pl.pallas_call(
        body,
        grid=(S, NB),
        in_specs=[pl.BlockSpec((1, RB, D), lambda s, j: (s, j, 0)),
                  pl.BlockSpec((1, 1, D), lambda s, j: (s, 0, 0))],
        out_specs=pl.BlockSpec((1, D, D), lambda s, j: (s, 0, 0)),
        out_shape=jax.ShapeDtypeStruct((S, D, D), jnp.float32),
    )(acc, mean)


def _proj_tc(acc, mean, comps):
    def body(a_ref, m_ref, c_ref, o_ref):
        yc = a_ref[0] - m_ref[0]
        o_ref[0] = lax.dot_general(yc, c_ref[0], (((1,), (0,)), ((), ())),
                                   preferred_element_type=jnp.float32)

    return pl.pallas_call(
        body,
        grid=(S, NB),
        in_specs=[pl.BlockSpec((1, RB, D), lambda s, j: (s, j, 0)),
                  pl.BlockSpec((1, 1, D), lambda s, j: (s, 0, 0)),
                  pl.BlockSpec((1, D, K), lambda s, j: (s, 0, 0))],
        out_specs=pl.BlockSpec((1, RB, K), lambda s, j: (s, j, 0)),
        out_shape=jax.ShapeDtypeStruct((S, NPAD, K), jnp.float32),
    )(acc, mean, comps)


def kernel(x, edge_index, coeffs, melt_embeddings=1, no_compression=0):
    x = x.astype(jnp.float32)
    src, dst = edge_index[0], edge_index[1]
    padlen = NTILES * EPT - E

    def prep(g, s):
        gp = jnp.concatenate([g, jnp.zeros((padlen,), jnp.int32)])
        sp = jnp.concatenate([s, jnp.full((padlen,), NPAD - 1, jnp.int32)])
        return (gp.reshape(NCORES, NSUB, CHUNKS, CHUNK),
                sp.reshape(NCORES, NSUB, CHUNKS, CHUNK))

    orient = (prep(src, dst), prep(dst, src))
    xpad = jnp.pad(x, ((0, NPAD - N), (0, 0)))
    ones = jnp.ones((NPAD, D), jnp.float32)
    zidx = jnp.zeros((NCORES, NSUB, CHUNKS, CHUNK), jnp.int32)

    outs = []
    for gat, sca in orient:
        dp = _spmm_sc(ones, zidx, sca)
        invb = _invdeg_tc(dp)
        y = xpad
        ys = [y]
        for _ in range(T):
            p = _spmm_sc(y, gat, sca)
            y = _combine_tc(p, invb)
            ys.append(y)
        acc = _acc_tc(ys, coeffs)
        mean = _mean_tc(acc)
        cov = _cov_tc(acc, mean)
        _, v = jnp.linalg.eigh(cov)
        comps = v[:, :, ::-1][:, :, :K]
        outs.append(_proj_tc(acc, mean, comps))

    emb = jnp.stack(outs, axis=0)
    emb = jnp.transpose(emb, (2, 0, 1, 3)).reshape(NPAD, 2 * S * K)[:N]
    scale = 1.0 + (jnp.asarray(no_compression)
                   * jnp.asarray(melt_embeddings)).astype(x.dtype)
    return emb * scale


# speedup vs baseline: 4.0561x; 4.0561x over previous
import functools

import jax
import jax.numpy as jnp
import numpy as np
from jax import lax
from jax.experimental import pallas as pl
from jax.experimental.pallas import tpu as pltpu
from jax.experimental.pallas import tpu_sc as plsc

N = 10000
E = 320000
D = 128
S = 4
T = 8
K = 16

NCORES = 2
NSUB = 16
NTILES = NCORES * NSUB
CHUNK = 128
CHUNKS = 81
CAP = CHUNKS * CHUNK
SIDE0 = N
TRASH = N + 32
OUTROWS = N + 112
ZPT = OUTROWS // NSUB
RBP = 400
NBP = N // RBP

_BH = [0] + [k * 10080 for k in range(1, 12)] + \
      [110880 + j * 9840 for j in range(1, 5)] + [160000]
BOUNDS = np.array(_BH + [160000 + v for v in _BH[1:]], np.int64)

_off = np.arange(CAP)[None, :]
_lo = BOUNDS[:-1][:, None]
_hi = BOUNDS[1:][:, None]
IDXMAT = np.minimum(_lo + _off, _hi - 1).astype(np.int32)
VALID = (_lo + _off) < _hi
PADGAT = ((np.arange(NTILES)[:, None] * CAP + _off) % N).astype(np.int32)
SIDE0_ROW = (SIDE0 + 2 * (np.arange(NTILES) % NSUB)).astype(np.int32)
SIDE1_ROW = SIDE0_ROW + 1


@functools.cache
def _mesh():
    return plsc.VectorSubcoreMesh(core_axis_name="c", subcore_axis_name="s",
                                  num_cores=NCORES, num_subcores=NSUB)


def _spmm_fold(y, gat, flg, srw):
    @functools.partial(
        pl.kernel,
        out_type=jax.ShapeDtypeStruct((NCORES, OUTROWS, D), jnp.float32),
        mesh=_mesh(),
        scratch_types=[
            pltpu.VMEM((CHUNKS, CHUNK), jnp.int32),
            pltpu.VMEM((CHUNKS, CHUNK), jnp.int32),
            pltpu.VMEM((CHUNKS, CHUNK), jnp.int32),
            pltpu.VMEM((CHUNK, D), jnp.float32),
            pltpu.VMEM((CHUNK, D), jnp.float32),
            pltpu.VMEM((CHUNK, D), jnp.float32),
            pltpu.VMEM((CHUNK, D), jnp.float32),
            pltpu.SemaphoreType.DMA,
            pltpu.SemaphoreType.DMA,
            pltpu.SemaphoreType.DMA,
        ],
    )
    def k(y_hbm, gat_hbm, flg_hbm, srw_hbm, out_hbm,
          gat_v, flg_v, srw_v, b0, b1, b2, st_v, s0, s1, s2):
        cid = lax.axis_index("c")
        sid = lax.axis_index("s")
        bufs = (b0, b1, b2)
        sems = (s0, s1, s2)

        zero16 = jnp.zeros((16,), jnp.float32)

        def zbody(r, carry):
            for c in range(D // 16):
                st_v[r, pl.ds(c * 16, 16)] = zero16
            return carry

        lax.fori_loop(0, CHUNK, zbody, 0)
        row0 = sid * ZPT
        for z in range(ZPT // CHUNK):
            pltpu.sync_copy(st_v, out_hbm.at[cid, pl.ds(row0 + z * CHUNK,
                                                        CHUNK)])
        rem = ZPT % CHUNK
        pltpu.sync_copy(st_v.at[pl.ds(0, rem)],
                        out_hbm.at[cid, pl.ds(row0 + ZPT - rem, rem)])
        plsc.subcore_barrier()

        pltpu.sync_copy(gat_hbm.at[cid, sid], gat_v)
        pltpu.sync_copy(flg_hbm.at[cid, sid], flg_v)
        pltpu.sync_copy(srw_hbm.at[cid, sid], srw_v)

        for kk in range(3):
            pltpu.async_copy(y_hbm.at[gat_v.at[kk]], bufs[kk], sems[kk])

        init = tuple(jnp.zeros((16,), jnp.float32) for _ in range(D // 16)) \
            + (jnp.int32(0),)

        def outer(i, carry):
            for kk in range(3):
                buf, sem = bufs[kk], sems[kk]
                g = i * 3 + kk
                pltpu.make_async_copy(y_hbm.at[gat_v.at[g]], buf, sem).wait()

                def inner(j, c2):
                    f16 = flg_v[g, pl.ds(pl.multiple_of(j * 16, 8), 16)]
                    for lane in range(16):
                        accs, p = c2[:-1], c2[-1]
                        e = j * 16 + lane
                        new = tuple(accs[c] + buf[e, pl.ds(c * 16, 16)]
                                    for c in range(D // 16))
                        f = f16[lane]
                        pm = p & 127
                        for c in range(D // 16):
                            st_v[pm, pl.ds(c * 16, 16)] = new[c]
                        pnew = p + f
                        do_flush = (f == 1) & ((pnew & 127) == 0)

                        @pl.when(do_flush)
                        def _():
                            ch = (pnew >> 7) - 1
                            pltpu.sync_copy(
                                st_v, out_hbm.at[cid].at[srw_v.at[ch]])

                        keep = f == 0
                        nxt = tuple(jnp.where(keep, new[c], zero16)
                                    for c in range(D // 16))
                        c2 = nxt + (pnew,)
                    return c2

                c2 = lax.fori_loop(0, CHUNK // 16, inner, carry)
                carry = c2

                @pl.when(g + 3 < CHUNKS)
                def _():
                    pltpu.async_copy(y_hbm.at[gat_v.at[g + 3]], buf, sem)
            return carry

        carry = lax.fori_loop(0, CHUNKS // 3, outer, init)
        p = carry[-1]

        @pl.when((p & 127) != 0)
        def _():
            pltpu.sync_copy(st_v, out_hbm.at[cid].at[srw_v.at[p >> 7]])

    return k(y, gat, flg, srw)


def _proj_tc(acc, mean, comps):
    def body(a_ref, m_ref, c_ref, o_ref):
        yc = a_ref[0] - m_ref[0]
        o_ref[0] = lax.dot_general(yc, c_ref[0], (((1,), (0,)), ((), ())),
                                   preferred_element_type=jnp.float32)

    return pl.pallas_call(
        body,
        grid=(S, NBP),
        in_specs=[pl.BlockSpec((1, RBP, D), lambda s, j: (s, j, 0)),
                  pl.BlockSpec((1, 1, D), lambda s, j: (s, 0, 0)),
                  pl.BlockSpec((1, D, K), lambda s, j: (s, 0, 0))],
        out_specs=pl.BlockSpec((1, RBP, K), lambda s, j: (s, j, 0)),
        out_shape=jax.ShapeDtypeStruct((S, N, K), jnp.float32),
    )(acc, mean, comps)


def _prep_orientation(e_src, e_dst):
    order = jnp.argsort(e_dst, stable=True)
    ss = e_src[order]
    ds_ = e_dst[order]

    idxmat = jnp.asarray(IDXMAT)
    valid = jnp.asarray(VALID)
    gat_t = jnp.where(valid, ss[idxmat], jnp.asarray(PADGAT))
    sca_t = jnp.where(valid, ds_[idxmat], TRASH)

    islast = jnp.concatenate(
        [sca_t[:, 1:] != sca_t[:, :-1],
         jnp.ones((NTILES, 1), bool)], axis=1)
    nseg_real = jnp.sum(islast & valid, axis=1).astype(jnp.int32)

    perm = jnp.argsort(~islast, axis=1, stable=True)
    seg_rows = jnp.take_along_axis(sca_t, perm, axis=1)
    pidx = jnp.arange(CAP)[None, :]
    srows = jnp.where(pidx >= nseg_real[:, None], TRASH, seg_rows)
    srows = jnp.where(pidx == (nseg_real - 1)[:, None],
                      jnp.asarray(SIDE1_ROW)[:, None], srows)
    srows = jnp.where(pidx == 0, jnp.asarray(SIDE0_ROW)[:, None], srows)

    def shape(a):
        return a.astype(jnp.int32).reshape(NCORES, NSUB, CHUNKS, CHUNK)

    b_lo = jnp.asarray(BOUNDS[:-1].astype(np.int32))
    b_hi = jnp.asarray((BOUNDS[1:] - 1).astype(np.int32))
    d_first = ds_[b_lo]
    d_last = ds_[b_hi]
    fixrows = jnp.stack([d_first, d_last], axis=1).reshape(-1)

    deg = (jnp.searchsorted(ds_, jnp.arange(N), side='right')
           - jnp.searchsorted(ds_, jnp.arange(N), side='left')
           ).astype(jnp.float32)

    return shape(gat_t), shape(islast), shape(srows), fixrows, deg


def kernel(x, edge_index, coeffs, melt_embeddings=1, no_compression=0):
    x = x.astype(jnp.float32)
    src, dst = edge_index[0], edge_index[1]
    weights = jax.nn.softmax(coeffs, axis=-1)

    outs = []
    for o in range(2):
        e_src, e_dst = (src, dst) if o == 0 else (dst, src)
        gat, flg, srw, fixrows, deg = _prep_orientation(e_src, e_dst)
        clipdeg = jnp.clip(deg, 1.0)[:, None]

        y = x
        ys = [x]
        for _ in range(T):
            p = _spmm_fold(y, gat, flg, srw)
            sidevals = p[:, SIDE0:SIDE0 + 2 * NSUB].reshape(-1, D)
            agg = (p[0, :N] + p[1, :N]).at[fixrows].add(sidevals)
            y = agg / clipdeg
            ys.append(y)

        acc = weights[:, 0][:, None, None] * x[None]
        for t in range(1, T + 1):
            acc = acc + weights[:, t][:, None, None] * ys[t][None]
        mean = acc.mean(axis=1, keepdims=True)
        yc = acc - mean
        cov = jnp.einsum('snd,sne->sde', yc, yc) / (N - 1)
        _, v = jnp.linalg.eigh(cov)
        comps = v[:, :, ::-1][:, :, :K]
        outs.append(_proj_tc(acc, mean, comps))

    emb = jnp.stack(outs, axis=0)
    emb = jnp.transpose(emb, (2, 0, 1, 3)).reshape(N, NCORES * S * K)
    scale = 1.0 + (jnp.asarray(no_compression)
                   * jnp.asarray(melt_embeddings)).astype(x.dtype)
    return emb * scale
